# Initial kernel scaffold; baseline (speedup 1.0000x reference)
import jax
import jax.numpy as jnp
from jax.experimental import pallas as pl


def kernel(x, edge_index, W1, b1, W2, b2):
    raise NotImplementedError("write your pallas kernel here")


# speedup vs baseline: 26.1605x; 26.1605x over previous
import functools

import jax
import jax.numpy as jnp
from jax import lax
from jax.experimental import pallas as pl
from jax.experimental.pallas import tpu as pltpu
import jax.experimental.pallas.tpu_sc as plsc

N = 10000
E = 320000
D_IN = 128
D_HID = 128
D_OUT = 16

NC = 2
NS = 16
NW = NC * NS
CH = 128
KCH = 79
EP = NW * KCH * CH
NPAD = N + 112
RPT = NPAD // NS
NPAD1 = 10240
RPT1 = NPAD1 // NS

_MESH = plsc.VectorSubcoreMesh(core_axis_name="c", subcore_axis_name="s")


def _deg_body(dst_hbm, ones_hbm, zeros_hbm, out_hbm, dstv, onev, acc, sem):
    c = lax.axis_index("c")
    s = lax.axis_index("s")
    w = s * NC + c
    pltpu.sync_copy(zeros_hbm, acc.at[pl.ds(pl.multiple_of(s * RPT1, 8), RPT1)])
    pltpu.sync_copy(ones_hbm, onev)
    pltpu.sync_copy(dst_hbm.at[w], dstv)
    plsc.subcore_barrier()

    def chunk(j, carry):
        pltpu.sync_copy(onev, acc.at[dstv.at[j]], add=True)
        return carry

    lax.fori_loop(0, KCH, chunk, 0)
    plsc.subcore_barrier()
    off = pl.multiple_of(c * NPAD1 + s * RPT1, 8)
    pltpu.sync_copy(acc.at[pl.ds(pl.multiple_of(s * RPT1, 8), RPT1)],
                    out_hbm.at[pl.ds(off, RPT1)])


_deg_call = pl.kernel(
    _deg_body,
    out_type=jax.ShapeDtypeStruct((NC * NPAD1,), jnp.float32),
    mesh=_MESH,
    scratch_types=[
        pltpu.VMEM((KCH, CH), jnp.int32),
        pltpu.VMEM((CH,), jnp.float32),
        pltpu.VMEM_SHARED((NPAD1,), jnp.float32),
        pltpu.SemaphoreType.DMA,
    ],
)


def _gs_body(hs_hbm, src_hbm, dst_hbm, zeros_hbm, out_hbm,
             srcv, dstv, rows, acc, sem):
    c = lax.axis_index("c")
    s = lax.axis_index("s")
    w = s * NC + c
    pltpu.sync_copy(zeros_hbm, acc.at[pl.ds(s * RPT, RPT)])
    pltpu.sync_copy(src_hbm.at[w], srcv)
    pltpu.sync_copy(dst_hbm.at[w], dstv)
    plsc.subcore_barrier()

    def chunk(j, carry):
        pltpu.async_copy(hs_hbm.at[srcv.at[j]], rows, sem).wait()
        pltpu.sync_copy(rows, acc.at[dstv.at[j]], add=True)
        return carry

    lax.fori_loop(0, KCH, chunk, 0)
    plsc.subcore_barrier()
    pltpu.sync_copy(acc.at[pl.ds(s * RPT, RPT)],
                    out_hbm.at[pl.ds(c * NPAD + s * RPT, RPT)])


def _make_gs_call(d):
    params = None
    if d % 128 != 0:
        params = pltpu.CompilerParams(use_tc_tiling_on_sc=False)
    return pl.kernel(
        _gs_body,
        out_type=jax.ShapeDtypeStruct((NC * NPAD, d), jnp.float32),
        mesh=_MESH,
        compiler_params=params,
        scratch_types=[
            pltpu.VMEM((KCH, CH), jnp.int32),
            pltpu.VMEM((KCH, CH), jnp.int32),
            pltpu.VMEM((CH, d), jnp.float32),
            pltpu.VMEM_SHARED((NPAD, d), jnp.float32),
            pltpu.SemaphoreType.DMA,
        ],
    )


_gs_call_hid = _make_gs_call(D_HID)
_gs_call_out = _make_gs_call(D_OUT)


RB = 1000


def _tc1_body(deg_ref, x_ref, w1_ref, dinv_ref, hs1_ref):
    deg = deg_ref[:, 0:1] + deg_ref[:, 1:2] + 1.0
    dinv = lax.rsqrt(deg)
    h = jnp.dot(x_ref[...], w1_ref[...], preferred_element_type=jnp.float32)
    dinv_ref[...] = dinv
    hs1_ref[...] = h * dinv


def _tc1(deg2, x, w1):
    return pl.pallas_call(
        _tc1_body,
        grid=(N // RB,),
        in_specs=[
            pl.BlockSpec((RB, 2), lambda i: (i, 0)),
            pl.BlockSpec((RB, D_IN), lambda i: (i, 0)),
            pl.BlockSpec((D_IN, D_HID), lambda i: (0, 0)),
        ],
        out_specs=[
            pl.BlockSpec((RB, 1), lambda i: (i, 0)),
            pl.BlockSpec((RB, D_HID), lambda i: (i, 0)),
        ],
        out_shape=[
            jax.ShapeDtypeStruct((N, 1), jnp.float32),
            jax.ShapeDtypeStruct((N, D_HID), jnp.float32),
        ],
    )(deg2, x, w1)


def _tc2_body(agga_ref, aggb_ref, hs1_ref, dinv_ref, b1_ref, w2_ref, hs2_ref):
    dinv = dinv_ref[...]
    z = (agga_ref[...] + aggb_ref[...] + hs1_ref[...]) * dinv
    g = jnp.maximum(z + b1_ref[...][None, :], 0.0)
    h2 = jnp.dot(g, w2_ref[...], preferred_element_type=jnp.float32)
    hs2_ref[...] = h2 * dinv


def _tc2(agga, aggb, hs1, dinv, b1, w2):
    return pl.pallas_call(
        _tc2_body,
        grid=(N // RB,),
        in_specs=[
            pl.BlockSpec((RB, D_HID), lambda i: (i, 0)),
            pl.BlockSpec((RB, D_HID), lambda i: (i, 0)),
            pl.BlockSpec((RB, D_HID), lambda i: (i, 0)),
            pl.BlockSpec((RB, 1), lambda i: (i, 0)),
            pl.BlockSpec((D_HID,), lambda i: (0,)),
            pl.BlockSpec((D_HID, D_OUT), lambda i: (0, 0)),
        ],
        out_specs=pl.BlockSpec((RB, D_OUT), lambda i: (i, 0)),
        out_shape=jax.ShapeDtypeStruct((N, D_OUT), jnp.float32),
    )(agga, aggb, hs1, dinv, b1, w2)


def _tc3_body(agga_ref, aggb_ref, hs2_ref, dinv_ref, b2_ref, out_ref):
    dinv = dinv_ref[...]
    z = (agga_ref[...] + aggb_ref[...] + hs2_ref[...]) * dinv
    z = z + b2_ref[...][None, :]
    m = jnp.max(z, axis=1, keepdims=True)
    out_ref[...] = z - m - jnp.log(jnp.sum(jnp.exp(z - m), axis=1,
                                           keepdims=True))


def _tc3(agga, aggb, hs2, dinv, b2):
    return pl.pallas_call(
        _tc3_body,
        grid=(N // RB,),
        in_specs=[
            pl.BlockSpec((RB, D_OUT), lambda i: (i, 0)),
            pl.BlockSpec((RB, D_OUT), lambda i: (i, 0)),
            pl.BlockSpec((RB, D_OUT), lambda i: (i, 0)),
            pl.BlockSpec((RB, 1), lambda i: (i, 0)),
            pl.BlockSpec((D_OUT,), lambda i: (0,)),
        ],
        out_specs=pl.BlockSpec((RB, D_OUT), lambda i: (i, 0)),
        out_shape=jax.ShapeDtypeStruct((N, D_OUT), jnp.float32),
    )(agga, aggb, hs2, dinv, b2)


def kernel(x, edge_index, W1, b1, W2, b2):
    ei = edge_index.astype(jnp.int32)
    pad = EP - E
    pidx = jnp.arange(pad, dtype=jnp.int32)
    src3 = jnp.concatenate([ei[0], pidx % N]).reshape(NW, KCH, CH)
    dst3 = jnp.concatenate([ei[1], N + (pidx % (NPAD - N))]).reshape(NW, KCH, CH)

    ones_ch = jnp.ones((CH,), jnp.float32)
    zeros1 = jnp.zeros((RPT1,), jnp.float32)
    zeros_hid = jnp.zeros((RPT, D_HID), jnp.float32)
    zeros_out = jnp.zeros((RPT, D_OUT), jnp.float32)

    degflat = _deg_call(dst3, ones_ch, zeros1)
    deg2 = jnp.stack([degflat[:N], degflat[NPAD1:NPAD1 + N]], axis=1)

    dinv, hs1 = _tc1(deg2, x, W1)

    agg1 = _gs_call_hid(hs1, src3, dst3, zeros_hid)
    hs2 = _tc2(agg1[:N], agg1[NPAD:NPAD + N], hs1, dinv, b1, W2)

    agg2 = _gs_call_out(hs2, src3, dst3, zeros_out)
    return _tc3(agg2[:N], agg2[NPAD:NPAD + N], hs2, dinv, b2)


# speedup vs baseline: 33.0095x; 1.2618x over previous
import functools

import jax
import jax.numpy as jnp
from jax import lax
from jax.experimental import pallas as pl
from jax.experimental.pallas import tpu as pltpu
import jax.experimental.pallas.tpu_sc as plsc

N = 10000
E = 320000
D_IN = 128
D_HID = 128
D_OUT = 16

NC = 2
NS = 16
NW = NC * NS
CH = 128
KCH = 80
EP = NW * KCH * CH
NPAD = N + 112
RPT = NPAD // NS
NPAD1 = 10240
RPT1 = NPAD1 // NS

_MESH = plsc.VectorSubcoreMesh(core_axis_name="c", subcore_axis_name="s")


def _deg_body(dst_hbm, ones_hbm, zeros_hbm, out_hbm, dstv, onev, acc, sem):
    c = lax.axis_index("c")
    s = lax.axis_index("s")
    w = s * NC + c
    pltpu.sync_copy(zeros_hbm, acc.at[pl.ds(pl.multiple_of(s * RPT1, 8), RPT1)])
    pltpu.sync_copy(ones_hbm, onev)
    pltpu.sync_copy(dst_hbm.at[w], dstv)
    plsc.subcore_barrier()

    def chunk(j, carry):
        pltpu.sync_copy(onev, acc.at[dstv.at[j]], add=True)
        return carry

    lax.fori_loop(0, KCH, chunk, 0)
    plsc.subcore_barrier()
    off = pl.multiple_of(c * NPAD1 + s * RPT1, 8)
    pltpu.sync_copy(acc.at[pl.ds(pl.multiple_of(s * RPT1, 8), RPT1)],
                    out_hbm.at[pl.ds(off, RPT1)])


_deg_call = pl.kernel(
    _deg_body,
    out_type=jax.ShapeDtypeStruct((NC * NPAD1,), jnp.float32),
    mesh=_MESH,
    scratch_types=[
        pltpu.VMEM((KCH, CH), jnp.int32),
        pltpu.VMEM((CH,), jnp.float32),
        pltpu.VMEM_SHARED((NPAD1,), jnp.float32),
        pltpu.SemaphoreType.DMA,
    ],
)


def _make_gs_body(sb, nh):
    kh = KCH // nh
    ksup = kh // sb

    def _gs_body(hs_hbm, src_hbm, dst_hbm, zeros_hbm, out_hbm,
                 srcv, dstv, rows0, rows1, acc, sg0, sg1, ss0, ss1):
        c = lax.axis_index("c")
        s = lax.axis_index("s")
        w = s * NC + c
        pltpu.sync_copy(zeros_hbm, acc.at[pl.ds(s * RPT, RPT)])
        plsc.subcore_barrier()

        def fire_g(buf, sem, sup):
            for t in range(sb):
                pltpu.async_copy(hs_hbm.at[srcv.at[sup * sb + t]],
                                 buf.at[pl.ds(t * CH, CH)], sem)

        def drain_g(buf, sem, sup):
            for t in range(sb):
                pltpu.make_async_copy(hs_hbm.at[srcv.at[sup * sb + t]],
                                      buf.at[pl.ds(t * CH, CH)], sem).wait()

        def fire_sc(buf, sem, sup):
            for t in range(sb):
                pltpu.async_copy(buf.at[pl.ds(t * CH, CH)],
                                 acc.at[dstv.at[sup * sb + t]], sem, add=True)

        def drain_sc(buf, sem, sup):
            for t in range(sb):
                pltpu.make_async_copy(buf.at[pl.ds(t * CH, CH)],
                                      acc.at[dstv.at[sup * sb + t]], sem).wait()

        def pair(jj, carry):
            j0 = 2 * jj
            j1 = j0 + 1
            fire_g(rows1, sg1, j1)
            drain_g(rows0, sg0, j0)
            fire_sc(rows0, ss0, j0)
            drain_g(rows1, sg1, j1)
            fire_sc(rows1, ss1, j1)
            drain_sc(rows0, ss0, j0)

            @pl.when(jj < ksup // 2 - 1)
            def _():
                fire_g(rows0, sg0, j0 + 2)

            drain_sc(rows1, ss1, j1)
            return carry

        for h in range(nh):
            pltpu.sync_copy(src_hbm.at[w, pl.ds(h * kh, kh)], srcv)
            pltpu.sync_copy(dst_hbm.at[w, pl.ds(h * kh, kh)], dstv)
            fire_g(rows0, sg0, 0)
            lax.fori_loop(0, ksup // 2, pair, 0)

        plsc.subcore_barrier()
        pltpu.sync_copy(acc.at[pl.ds(s * RPT, RPT)],
                        out_hbm.at[pl.ds(c * NPAD + s * RPT, RPT)])

    return _gs_body


def _make_gs_call(d, sb, nh):
    params = None
    if d % 128 != 0:
        params = pltpu.CompilerParams(use_tc_tiling_on_sc=False)
    return pl.kernel(
        _make_gs_body(sb, nh),
        out_type=jax.ShapeDtypeStruct((NC * NPAD, d), jnp.float32),
        mesh=_MESH,
        compiler_params=params,
        scratch_types=[
            pltpu.VMEM((KCH // nh, CH), jnp.int32),
            pltpu.VMEM((KCH // nh, CH), jnp.int32),
            pltpu.VMEM((sb * CH, d), jnp.float32),
            pltpu.VMEM((sb * CH, d), jnp.float32),
            pltpu.VMEM_SHARED((NPAD, d), jnp.float32),
            pltpu.SemaphoreType.DMA,
            pltpu.SemaphoreType.DMA,
            pltpu.SemaphoreType.DMA,
            pltpu.SemaphoreType.DMA,
        ],
    )


_gs_call_hid = _make_gs_call(D_HID, 1, 2)
_gs_call_out = _make_gs_call(D_OUT, 4, 1)


RB = 1000


def _tc1_body(deg_ref, x_ref, w1_ref, dinv_ref, hs1_ref):
    deg = deg_ref[:, 0:1] + deg_ref[:, 1:2] + 1.0
    dinv = lax.rsqrt(deg)
    h = jnp.dot(x_ref[...], w1_ref[...], preferred_element_type=jnp.float32)
    dinv_ref[...] = dinv
    hs1_ref[...] = h * dinv


def _tc1(deg2, x, w1):
    return pl.pallas_call(
        _tc1_body,
        grid=(N // RB,),
        in_specs=[
            pl.BlockSpec((RB, 2), lambda i: (i, 0)),
            pl.BlockSpec((RB, D_IN), lambda i: (i, 0)),
            pl.BlockSpec((D_IN, D_HID), lambda i: (0, 0)),
        ],
        out_specs=[
            pl.BlockSpec((RB, 1), lambda i: (i, 0)),
            pl.BlockSpec((RB, D_HID), lambda i: (i, 0)),
        ],
        out_shape=[
            jax.ShapeDtypeStruct((N, 1), jnp.float32),
            jax.ShapeDtypeStruct((N, D_HID), jnp.float32),
        ],
    )(deg2, x, w1)


def _tc2_body(agga_ref, aggb_ref, hs1_ref, dinv_ref, b1_ref, w2_ref, hs2_ref):
    dinv = dinv_ref[...]
    z = (agga_ref[...] + aggb_ref[...] + hs1_ref[...]) * dinv
    g = jnp.maximum(z + b1_ref[...][None, :], 0.0)
    h2 = jnp.dot(g, w2_ref[...], preferred_element_type=jnp.float32)
    hs2_ref[...] = h2 * dinv


def _tc2(agga, aggb, hs1, dinv, b1, w2):
    return pl.pallas_call(
        _tc2_body,
        grid=(N // RB,),
        in_specs=[
            pl.BlockSpec((RB, D_HID), lambda i: (i, 0)),
            pl.BlockSpec((RB, D_HID), lambda i: (i, 0)),
            pl.BlockSpec((RB, D_HID), lambda i: (i, 0)),
            pl.BlockSpec((RB, 1), lambda i: (i, 0)),
            pl.BlockSpec((D_HID,), lambda i: (0,)),
            pl.BlockSpec((D_HID, D_OUT), lambda i: (0, 0)),
        ],
        out_specs=pl.BlockSpec((RB, D_OUT), lambda i: (i, 0)),
        out_shape=jax.ShapeDtypeStruct((N, D_OUT), jnp.float32),
    )(agga, aggb, hs1, dinv, b1, w2)


def _tc3_body(agga_ref, aggb_ref, hs2_ref, dinv_ref, b2_ref, out_ref):
    dinv = dinv_ref[...]
    z = (agga_ref[...] + aggb_ref[...] + hs2_ref[...]) * dinv
    z = z + b2_ref[...][None, :]
    m = jnp.max(z, axis=1, keepdims=True)
    out_ref[...] = z - m - jnp.log(jnp.sum(jnp.exp(z - m), axis=1,
                                           keepdims=True))


def _tc3(agga, aggb, hs2, dinv, b2):
    return pl.pallas_call(
        _tc3_body,
        grid=(N // RB,),
        in_specs=[
            pl.BlockSpec((RB, D_OUT), lambda i: (i, 0)),
            pl.BlockSpec((RB, D_OUT), lambda i: (i, 0)),
            pl.BlockSpec((RB, D_OUT), lambda i: (i, 0)),
            pl.BlockSpec((RB, 1), lambda i: (i, 0)),
            pl.BlockSpec((D_OUT,), lambda i: (0,)),
        ],
        out_specs=pl.BlockSpec((RB, D_OUT), lambda i: (i, 0)),
        out_shape=jax.ShapeDtypeStruct((N, D_OUT), jnp.float32),
    )(agga, aggb, hs2, dinv, b2)


def kernel(x, edge_index, W1, b1, W2, b2):
    ei = edge_index.astype(jnp.int32)
    pad = EP - E
    pidx = jnp.arange(pad, dtype=jnp.int32)
    src3 = jnp.concatenate([ei[0], pidx % N]).reshape(NW, KCH, CH)
    dst3 = jnp.concatenate([ei[1], N + (pidx % (NPAD - N))]).reshape(NW, KCH, CH)

    ones_ch = jnp.ones((CH,), jnp.float32)
    zeros1 = jnp.zeros((RPT1,), jnp.float32)
    zeros_hid = jnp.zeros((RPT, D_HID), jnp.float32)
    zeros_out = jnp.zeros((RPT, D_OUT), jnp.float32)

    degflat = _deg_call(dst3, ones_ch, zeros1)
    deg2 = jnp.stack([degflat[:N], degflat[NPAD1:NPAD1 + N]], axis=1)

    dinv, hs1 = _tc1(deg2, x, W1)

    agg1 = _gs_call_hid(hs1, src3, dst3, zeros_hid)
    hs2 = _tc2(agg1[:N], agg1[NPAD:NPAD + N], hs1, dinv, b1, W2)

    agg2 = _gs_call_out(hs2, src3, dst3, zeros_out)
    return _tc3(agg2[:N], agg2[NPAD:NPAD + N], hs2, dinv, b2)


# speedup vs baseline: 35.4558x; 1.0741x over previous
import jax
import jax.numpy as jnp
from jax import lax
from jax.experimental import pallas as pl
from jax.experimental.pallas import tpu as pltpu
import jax.experimental.pallas.tpu_sc as plsc

N = 10000
E = 320000
D_IN = 128
D_HID = 128
D_OUT = 16

NC = 2
NS = 16
NW = NC * NS
CH = 128
KCH = 80
EP = NW * KCH * CH
NPAD = N + 112
RPT = NPAD // NS
NPAD1 = 10240
RPT1 = NPAD1 // NS

_MESH = plsc.VectorSubcoreMesh(core_axis_name="c", subcore_axis_name="s")


def _deg_body(dst_hbm, ones_hbm, zeros_hbm, outa_hbm, outb_hbm,
              dstv, onev, acc, sem):
    c = lax.axis_index("c")
    s = lax.axis_index("s")
    w = s * NC + c
    pltpu.sync_copy(zeros_hbm, acc.at[pl.ds(pl.multiple_of(s * RPT1, 8), RPT1)])
    pltpu.sync_copy(ones_hbm, onev)
    pltpu.sync_copy(dst_hbm.at[w], dstv)
    plsc.subcore_barrier()

    def chunk(j, carry):
        pltpu.sync_copy(onev, acc.at[dstv.at[j]], add=True)
        return carry

    lax.fori_loop(0, KCH, chunk, 0)
    plsc.subcore_barrier()
    sl = pl.ds(pl.multiple_of(s * RPT1, 8), RPT1)

    @pl.when(c == 0)
    def _():
        pltpu.sync_copy(acc.at[sl], outa_hbm.at[sl])

    @pl.when(c == 1)
    def _():
        pltpu.sync_copy(acc.at[sl], outb_hbm.at[sl])


_deg_call = pl.kernel(
    _deg_body,
    out_type=[jax.ShapeDtypeStruct((NPAD1,), jnp.float32),
              jax.ShapeDtypeStruct((NPAD1,), jnp.float32)],
    mesh=_MESH,
    scratch_types=[
        pltpu.VMEM((KCH, CH), jnp.int32),
        pltpu.VMEM((CH,), jnp.float32),
        pltpu.VMEM_SHARED((NPAD1,), jnp.float32),
        pltpu.SemaphoreType.DMA,
    ],
)


def _make_gs_body(sb, nh):
    kh = KCH // nh
    ksup = kh // sb

    def _gs_body(hs_hbm, src_hbm, dst_hbm, zeros_hbm, outa_hbm, outb_hbm,
                 srcv, dstv, rows0, rows1, acc, sg0, sg1, ss0, ss1):
        c = lax.axis_index("c")
        s = lax.axis_index("s")
        w = s * NC + c
        pltpu.sync_copy(zeros_hbm, acc.at[pl.ds(s * RPT, RPT)])
        plsc.subcore_barrier()

        def fire_g(buf, sem, sup):
            for t in range(sb):
                pltpu.async_copy(hs_hbm.at[srcv.at[sup * sb + t]],
                                 buf.at[pl.ds(t * CH, CH)], sem)

        def drain_g(buf, sem, sup):
            for t in range(sb):
                pltpu.make_async_copy(hs_hbm.at[srcv.at[sup * sb + t]],
                                      buf.at[pl.ds(t * CH, CH)], sem).wait()

        def fire_sc(buf, sem, sup):
            for t in range(sb):
                pltpu.async_copy(buf.at[pl.ds(t * CH, CH)],
                                 acc.at[dstv.at[sup * sb + t]], sem, add=True)

        def drain_sc(buf, sem, sup):
            for t in range(sb):
                pltpu.make_async_copy(buf.at[pl.ds(t * CH, CH)],
                                      acc.at[dstv.at[sup * sb + t]], sem).wait()

        def pair(jj, carry):
            j0 = 2 * jj
            j1 = j0 + 1
            fire_g(rows1, sg1, j1)
            drain_g(rows0, sg0, j0)
            fire_sc(rows0, ss0, j0)
            drain_g(rows1, sg1, j1)
            fire_sc(rows1, ss1, j1)
            drain_sc(rows0, ss0, j0)

            @pl.when(jj < ksup // 2 - 1)
            def _():
                fire_g(rows0, sg0, j0 + 2)

            drain_sc(rows1, ss1, j1)
            return carry

        for h in range(nh):
            pltpu.sync_copy(src_hbm.at[w, pl.ds(h * kh, kh)], srcv)
            pltpu.sync_copy(dst_hbm.at[w, pl.ds(h * kh, kh)], dstv)
            fire_g(rows0, sg0, 0)
            lax.fori_loop(0, ksup // 2, pair, 0)

        plsc.subcore_barrier()
        sl = pl.ds(pl.multiple_of(s * RPT, 8), RPT)

        @pl.when(c == 0)
        def _():
            pltpu.sync_copy(acc.at[sl], outa_hbm.at[sl])

        @pl.when(c == 1)
        def _():
            pltpu.sync_copy(acc.at[sl], outb_hbm.at[sl])

    return _gs_body


def _make_gs_call(d, sb, nh):
    params = None
    if d % 128 != 0:
        params = pltpu.CompilerParams(use_tc_tiling_on_sc=False)
    out = jax.ShapeDtypeStruct((NPAD, d), jnp.float32)
    return pl.kernel(
        _make_gs_body(sb, nh),
        out_type=[out, out],
        mesh=_MESH,
        compiler_params=params,
        scratch_types=[
            pltpu.VMEM((KCH // nh, CH), jnp.int32),
            pltpu.VMEM((KCH // nh, CH), jnp.int32),
            pltpu.VMEM((sb * CH, d), jnp.float32),
            pltpu.VMEM((sb * CH, d), jnp.float32),
            pltpu.VMEM_SHARED((NPAD, d), jnp.float32),
            pltpu.SemaphoreType.DMA,
            pltpu.SemaphoreType.DMA,
            pltpu.SemaphoreType.DMA,
            pltpu.SemaphoreType.DMA,
        ],
    )


_gs_call_hid = _make_gs_call(D_HID, 1, 2)
_gs_call_out = _make_gs_call(D_OUT, 4, 1)


RB = 1000


def _tc1a_body(x_ref, w1_ref, h1_ref):
    h1_ref[...] = jnp.dot(x_ref[...], w1_ref[...],
                          preferred_element_type=jnp.float32)


def _tc1a(x, w1):
    return pl.pallas_call(
        _tc1a_body,
        grid=(N // RB,),
        in_specs=[
            pl.BlockSpec((RB, D_IN), lambda i: (i, 0)),
            pl.BlockSpec((D_IN, D_HID), lambda i: (0, 0)),
        ],
        out_specs=pl.BlockSpec((RB, D_HID), lambda i: (i, 0)),
        out_shape=jax.ShapeDtypeStruct((N, D_HID), jnp.float32),
    )(x, w1)


def _tc1b_body(dega_ref, degb_ref, h1_ref, dinv_ref, hs1_ref):
    deg = dega_ref[pl.ds(0, N)] + degb_ref[pl.ds(0, N)] + 1.0
    dinv = lax.rsqrt(deg)[:, None]
    dinv_ref[...] = dinv
    hs1_ref[...] = h1_ref[...] * dinv


def _tc1b(dega, degb, h1):
    return pl.pallas_call(
        _tc1b_body,
        grid=(1,),
        in_specs=[
            pl.BlockSpec((NPAD1,), lambda i: (0,)),
            pl.BlockSpec((NPAD1,), lambda i: (0,)),
            pl.BlockSpec((N, D_HID), lambda i: (0, 0)),
        ],
        out_specs=[
            pl.BlockSpec((N, 1), lambda i: (0, 0)),
            pl.BlockSpec((N, D_HID), lambda i: (0, 0)),
        ],
        out_shape=[
            jax.ShapeDtypeStruct((N, 1), jnp.float32),
            jax.ShapeDtypeStruct((N, D_HID), jnp.float32),
        ],
    )(dega, degb, h1)


def _tc2_body(agga_ref, aggb_ref, hs1_ref, dinv_ref, b1_ref, w2_ref, hs2_ref):
    dinv = dinv_ref[...]
    z = (agga_ref[...] + aggb_ref[...] + hs1_ref[...]) * dinv
    g = jnp.maximum(z + b1_ref[...][None, :], 0.0)
    h2 = jnp.dot(g, w2_ref[...], preferred_element_type=jnp.float32)
    hs2_ref[...] = h2 * dinv


def _tc2(agga, aggb, hs1, dinv, b1, w2):
    return pl.pallas_call(
        _tc2_body,
        grid=(N // RB,),
        in_specs=[
            pl.BlockSpec((RB, D_HID), lambda i: (i, 0)),
            pl.BlockSpec((RB, D_HID), lambda i: (i, 0)),
            pl.BlockSpec((RB, D_HID), lambda i: (i, 0)),
            pl.BlockSpec((RB, 1), lambda i: (i, 0)),
            pl.BlockSpec((D_HID,), lambda i: (0,)),
            pl.BlockSpec((D_HID, D_OUT), lambda i: (0, 0)),
        ],
        out_specs=pl.BlockSpec((RB, D_OUT), lambda i: (i, 0)),
        out_shape=jax.ShapeDtypeStruct((N, D_OUT), jnp.float32),
    )(agga, aggb, hs1, dinv, b1, w2)


def _tc3_body(agga_ref, aggb_ref, hs2_ref, dinv_ref, b2_ref, out_ref):
    dinv = dinv_ref[...]
    z = (agga_ref[...] + aggb_ref[...] + hs2_ref[...]) * dinv
    z = z + b2_ref[...][None, :]
    m = jnp.max(z, axis=1, keepdims=True)
    out_ref[...] = z - m - jnp.log(jnp.sum(jnp.exp(z - m), axis=1,
                                           keepdims=True))


def _tc3(agga, aggb, hs2, dinv, b2):
    return pl.pallas_call(
        _tc3_body,
        grid=(N // RB,),
        in_specs=[
            pl.BlockSpec((RB, D_OUT), lambda i: (i, 0)),
            pl.BlockSpec((RB, D_OUT), lambda i: (i, 0)),
            pl.BlockSpec((RB, D_OUT), lambda i: (i, 0)),
            pl.BlockSpec((RB, 1), lambda i: (i, 0)),
            pl.BlockSpec((D_OUT,), lambda i: (0,)),
        ],
        out_specs=pl.BlockSpec((RB, D_OUT), lambda i: (i, 0)),
        out_shape=jax.ShapeDtypeStruct((N, D_OUT), jnp.float32),
    )(agga, aggb, hs2, dinv, b2)


def kernel(x, edge_index, W1, b1, W2, b2):
    ei = edge_index.astype(jnp.int32)
    pad = EP - E
    pidx = jnp.arange(pad, dtype=jnp.int32)
    src3 = jnp.concatenate([ei[0], pidx % N]).reshape(NW, KCH, CH)
    dst3 = jnp.concatenate([ei[1], N + (pidx % (NPAD - N))]).reshape(NW, KCH, CH)

    ones_ch = jnp.ones((CH,), jnp.float32)
    zeros1 = jnp.zeros((RPT1,), jnp.float32)
    zeros_hid = jnp.zeros((RPT, D_HID), jnp.float32)
    zeros_out = jnp.zeros((RPT, D_OUT), jnp.float32)

    dega, degb = _deg_call(dst3, ones_ch, zeros1)
    h1 = _tc1a(x, W1)
    dinv, hs1 = _tc1b(dega, degb, h1)

    agg1a, agg1b = _gs_call_hid(hs1, src3, dst3, zeros_hid)
    hs2 = _tc2(agg1a, agg1b, hs1, dinv, b1, W2)

    agg2a, agg2b = _gs_call_out(hs2, src3, dst3, zeros_out)
    return _tc3(agg2a, agg2b, hs2, dinv, b2)
